# Initial kernel scaffold; baseline (speedup 1.0000x reference)
import jax
import jax.numpy as jnp
from jax.experimental import pallas as pl


def kernel(weight_hh, distance_matrix):
    raise NotImplementedError("write your pallas kernel here")


# speedup vs baseline: 61.9668x; 61.9668x over previous
import jax
import jax.numpy as jnp
from jax.experimental import pallas as pl
from jax.experimental.pallas import tpu as pltpu

N = 4096
NUM_BINS = 30
LAMBD = 0.01

_RB_MM = 512
_RB_H = 256


def _minmax_body(d_ref, o_ref, mn_ref, mx_ref):
    i = pl.program_id(0)

    @pl.when(i == 0)
    def _():
        mn_ref[0] = jnp.float32(jnp.inf)
        mx_ref[0] = jnp.float32(-jnp.inf)

    d = d_ref[...]
    mn_ref[0] = jnp.minimum(mn_ref[0], jnp.min(d))
    mx_ref[0] = jnp.maximum(mx_ref[0], jnp.max(d))

    @pl.when(i == pl.num_programs(0) - 1)
    def _():
        o_ref[0] = mn_ref[0]
        o_ref[1] = mx_ref[0]


_minmax = pl.pallas_call(
    _minmax_body,
    grid=(N // _RB_MM,),
    in_specs=[pl.BlockSpec((_RB_MM, N), lambda i: (i, 0))],
    out_specs=pl.BlockSpec(memory_space=pltpu.SMEM),
    out_shape=jax.ShapeDtypeStruct((2,), jnp.float32),
    scratch_shapes=[pltpu.SMEM((1,), jnp.float32),
                    pltpu.SMEM((1,), jnp.float32)],
)


def _hist_body(w_ref, d_ref, bins_ref, o_ref, acc_ref):
    i = pl.program_id(0)

    @pl.when(i == 0)
    def _():
        acc_ref[...] = jnp.zeros_like(acc_ref)

    a = jnp.abs(w_ref[...])
    d = d_ref[...]
    for j in range(NUM_BINS + 1):
        m = jnp.where(d > bins_ref[j], a, 0.0)
        acc_ref[j, :] += jnp.sum(m, axis=0)

    @pl.when(i == pl.num_programs(0) - 1)
    def _():
        s = jnp.sum(acc_ref[...], axis=1)
        t = s[:NUM_BINS] - s[1:]
        total = s[0] - s[NUM_BINS] + 1e-8
        p = t / total
        neg_ent = jnp.sum(p * jnp.log(p + 1e-8))
        o_ref[0] = LAMBD * neg_ent


_hist = pl.pallas_call(
    _hist_body,
    grid=(N // _RB_H,),
    in_specs=[
        pl.BlockSpec((_RB_H, N), lambda i: (i, 0)),
        pl.BlockSpec((_RB_H, N), lambda i: (i, 0)),
        pl.BlockSpec(memory_space=pltpu.SMEM),
    ],
    out_specs=pl.BlockSpec(memory_space=pltpu.SMEM),
    out_shape=jax.ShapeDtypeStruct((1,), jnp.float32),
    scratch_shapes=[pltpu.VMEM((NUM_BINS + 1, N), jnp.float32)],
)


def kernel(weight_hh, distance_matrix):
    mm = _minmax(distance_matrix)
    bins = jnp.linspace(mm[0], mm[1], NUM_BINS + 1)
    loss = _hist(weight_hh, distance_matrix, bins)
    return loss[0]
